# baseline (speedup 1.0000x reference)
import functools
import jax
import jax.numpy as jnp
from jax import lax
from jax.experimental import pallas as pl
from jax.experimental.pallas import tpu as pltpu
from jax.experimental.pallas import tpu_sc as plsc

IN_CH = 256
IN_LEN = 131072
OUT_LEN = 120423
CHUNK = 3072
NFULL = OUT_LEN // CHUNK
LANES = 16
NC, NS = 2, 16
NW = NC * NS
CPW = IN_CH // NW
SPAN = 3360


def _g0_of(j0):
    return (j0 + (j0 * 10649) // 120422) // 8 * 8


BULK = NFULL * CHUNK
TAIL_G0 = _g0_of(BULK)
TAIL_SPAN = IN_LEN - TAIL_G0
TAIL_OUT = 640
TAIL_NVEC = TAIL_OUT // LANES
PAD_IDX = BULK + TAIL_OUT


@functools.partial(
    pl.kernel,
    out_type=jax.ShapeDtypeStruct((IN_CH, PAD_IDX), jnp.float32),
    mesh=plsc.VectorSubcoreMesh(core_axis_name="c", subcore_axis_name="s"),
    scratch_types=[
        pltpu.VMEM((2 * CPW, SPAN), jnp.float32),
        pltpu.VMEM((2 * CHUNK,), jnp.int32),
        pltpu.VMEM((2 * CHUNK,), jnp.float32),
        pltpu.VMEM((2 * CPW, CHUNK), jnp.float32),
        pltpu.SemaphoreType.DMA,
        pltpu.SemaphoreType.DMA,
        pltpu.SemaphoreType.DMA,
        pltpu.SemaphoreType.DMA,
    ],
    compiler_params=pltpu.CompilerParams(use_tc_tiling_on_sc=False,
                                         needs_layout_passes=False),
)
def _resample(x_hbm, y0_hbm, frac_hbm, bulk_hbm, xbuf, y0buf,
              fracbuf, outbuf, in_a, in_b, out_a, out_b):
    wid = lax.axis_index("s") * NC + lax.axis_index("c")
    cbase = wid * CPW
    rows0 = [jnp.full((LANES,), ci, jnp.int32) for ci in range(CPW)]
    in_sems = (in_a, in_b)
    out_sems = (out_a, out_b)

    def fire_in(k, p, sem):
        j0 = k * CHUNK
        g0 = _g0_of(j0)
        pltpu.async_copy(y0_hbm.at[pl.ds(j0, CHUNK)],
                         y0buf.at[pl.ds(p * CHUNK, CHUNK)], sem)
        pltpu.async_copy(frac_hbm.at[pl.ds(j0, CHUNK)],
                         fracbuf.at[pl.ds(p * CHUNK, CHUNK)], sem)
        pltpu.async_copy(x_hbm.at[pl.ds(cbase, CPW), pl.ds(g0, SPAN)],
                         xbuf.at[pl.ds(p * CPW, CPW), :], sem)

    def wait_in(p, sem):
        pltpu.make_async_copy(y0_hbm.at[pl.ds(0, CHUNK)],
                              y0buf.at[pl.ds(p * CHUNK, CHUNK)], sem).wait()
        pltpu.make_async_copy(frac_hbm.at[pl.ds(0, CHUNK)],
                              fracbuf.at[pl.ds(p * CHUNK, CHUNK)], sem).wait()
        pltpu.make_async_copy(x_hbm.at[pl.ds(0, CPW), pl.ds(0, SPAN)],
                              xbuf.at[pl.ds(p * CPW, CPW), :], sem).wait()

    def fire_out(k, p, sem):
        pltpu.async_copy(outbuf.at[pl.ds(p * CPW, CPW), :],
                         bulk_hbm.at[pl.ds(cbase, CPW), pl.ds(k * CHUNK, CHUNK)],
                         sem)

    def wait_out(p, sem):
        pltpu.make_async_copy(outbuf.at[pl.ds(p * CPW, CPW), :],
                              bulk_hbm.at[pl.ds(cbase, CPW), pl.ds(0, CHUNK)],
                              sem).wait()

    def compute(p, g0):
        roff = p * CPW
        rows = [rv + roff for rv in rows0]
        coff = p * CHUNK

        @plsc.parallel_loop(0, CHUNK, step=LANES, unroll=8)
        def vec_body(col):
            sl = pl.ds(coff + col, LANES)
            li0 = y0buf[sl] - g0
            li1 = li0 + 1
            fv = fracbuf[sl]
            for ci in range(CPW):
                v0 = plsc.load_gather(xbuf, [rows[ci], li0])
                v1 = plsc.load_gather(xbuf, [rows[ci], li1])
                outbuf[roff + ci, pl.ds(col, LANES)] = v0 + fv * (v1 - v0)

    fire_in(0, 0, in_a)
    fire_in(1, 1, in_b)
    wait_in(0, in_a)
    compute(0, _g0_of(0))
    fire_in(2, 0, in_a)
    fire_out(0, 0, out_a)
    wait_in(1, in_b)
    compute(1, _g0_of(CHUNK))
    fire_in(3, 1, in_b)
    fire_out(1, 1, out_b)

    def run(k, p):
        wait_out(p, out_sems[p])
        wait_in(p, in_sems[p])
        compute(p, _g0_of(k * CHUNK))
        kn = jnp.minimum(k + 2, NFULL - 1)
        fire_in(kn, p, in_sems[p])
        fire_out(k, p, out_sems[p])

    def pair_body(m, _):
        k = 2 + 2 * m
        run(k, 0)
        run(k + 1, 1)
        return 0

    lax.fori_loop(0, (NFULL - 3) // 2, pair_body, 0)
    run(NFULL - 1, 0)

    wait_out(1, out_b)
    wait_out(0, out_a)
    wait_in(1, in_b)
    wait_in(0, in_a)

    pltpu.sync_copy(y0_hbm.at[pl.ds(BULK, TAIL_OUT)], y0buf.at[pl.ds(0, TAIL_OUT)])
    pltpu.sync_copy(frac_hbm.at[pl.ds(BULK, TAIL_OUT)],
                    fracbuf.at[pl.ds(0, TAIL_OUT)])
    pltpu.sync_copy(x_hbm.at[pl.ds(cbase, CPW), pl.ds(TAIL_G0, TAIL_SPAN)],
                    xbuf.at[pl.ds(0, CPW), pl.ds(0, TAIL_SPAN)])
    li_max = IN_LEN - 1 - TAIL_G0

    @plsc.parallel_loop(0, TAIL_OUT, step=LANES, unroll=4)
    def tail_body(col):
        sl = pl.ds(col, LANES)
        li0 = y0buf[sl] - TAIL_G0
        li1 = jnp.minimum(li0 + 1, li_max)
        fv = fracbuf[sl]
        for ci in range(CPW):
            v0 = plsc.load_gather(xbuf, [rows0[ci], li0])
            v1 = plsc.load_gather(xbuf, [rows0[ci], li1])
            outbuf[ci, pl.ds(col, LANES)] = v0 + fv * (v1 - v0)

    pltpu.sync_copy(outbuf.at[pl.ds(0, CPW), pl.ds(0, TAIL_OUT)],
                    bulk_hbm.at[pl.ds(cbase, CPW), pl.ds(BULK, TAIL_OUT)])


def kernel(x, x_frac, y0_idx, y1_idx):
    pad = PAD_IDX - OUT_LEN
    y0p = jnp.pad(y0_idx, (0, pad), constant_values=IN_LEN - 2)
    fracp = jnp.pad(x_frac, (0, pad), constant_values=0.0)
    return _resample(x, y0p, fracp)[:, :OUT_LEN]


# speedup vs baseline: 1.7915x; 1.7915x over previous
import functools
import jax
import jax.numpy as jnp
from jax import lax
from jax.experimental import pallas as pl
from jax.experimental.pallas import tpu as pltpu
from jax.experimental.pallas import tpu_sc as plsc

IN_CH = 256
IN_LEN = 131072
OUT_LEN = 120423
CHUNK = 3072
NFULL = OUT_LEN // CHUNK
LANES = 16
NC, NS = 2, 16
NW = NC * NS
CPW = IN_CH // NW
SPAN = 3584


def _g0_of(j0):
    return (j0 + (j0 * 10649) // 120422) // 128 * 128


BULK = NFULL * CHUNK
TAIL_G0 = _g0_of(BULK)
TAIL_SPAN = IN_LEN - TAIL_G0
TAIL_OUT = 640
TAIL_NVEC = TAIL_OUT // LANES
PAD_IDX = BULK + TAIL_OUT


@functools.partial(
    pl.kernel,
    out_type=jax.ShapeDtypeStruct((IN_CH, PAD_IDX), jnp.float32),
    mesh=plsc.VectorSubcoreMesh(core_axis_name="c", subcore_axis_name="s"),
    scratch_types=[
        pltpu.VMEM((2 * CPW, SPAN), jnp.float32),
        pltpu.VMEM((2 * CHUNK,), jnp.int32),
        pltpu.VMEM((2 * CHUNK,), jnp.float32),
        pltpu.VMEM((2 * CPW, CHUNK), jnp.float32),
        pltpu.SemaphoreType.DMA,
        pltpu.SemaphoreType.DMA,
        pltpu.SemaphoreType.DMA,
        pltpu.SemaphoreType.DMA,
    ],
    compiler_params=pltpu.CompilerParams(use_tc_tiling_on_sc=True,
                                         needs_layout_passes=False),
)
def _resample(x_hbm, y0_hbm, frac_hbm, bulk_hbm, xbuf, y0buf,
              fracbuf, outbuf, in_a, in_b, out_a, out_b):
    wid = lax.axis_index("s") * NC + lax.axis_index("c")
    cbase = wid * CPW
    rows0 = [jnp.full((LANES,), ci, jnp.int32) for ci in range(CPW)]
    in_sems = (in_a, in_b)
    out_sems = (out_a, out_b)

    def fire_in(k, p, sem):
        j0 = k * CHUNK
        g0 = _g0_of(j0)
        pltpu.async_copy(y0_hbm.at[pl.ds(j0, CHUNK)],
                         y0buf.at[pl.ds(p * CHUNK, CHUNK)], sem)
        pltpu.async_copy(frac_hbm.at[pl.ds(j0, CHUNK)],
                         fracbuf.at[pl.ds(p * CHUNK, CHUNK)], sem)
        pltpu.async_copy(x_hbm.at[pl.ds(cbase, CPW), pl.ds(g0, SPAN)],
                         xbuf.at[pl.ds(p * CPW, CPW), :], sem)

    def wait_in(p, sem):
        pltpu.make_async_copy(y0_hbm.at[pl.ds(0, CHUNK)],
                              y0buf.at[pl.ds(p * CHUNK, CHUNK)], sem).wait()
        pltpu.make_async_copy(frac_hbm.at[pl.ds(0, CHUNK)],
                              fracbuf.at[pl.ds(p * CHUNK, CHUNK)], sem).wait()
        pltpu.make_async_copy(x_hbm.at[pl.ds(0, CPW), pl.ds(0, SPAN)],
                              xbuf.at[pl.ds(p * CPW, CPW), :], sem).wait()

    def fire_out(k, p, sem):
        pltpu.async_copy(outbuf.at[pl.ds(p * CPW, CPW), :],
                         bulk_hbm.at[pl.ds(cbase, CPW), pl.ds(k * CHUNK, CHUNK)],
                         sem)

    def wait_out(p, sem):
        pltpu.make_async_copy(outbuf.at[pl.ds(p * CPW, CPW), :],
                              bulk_hbm.at[pl.ds(cbase, CPW), pl.ds(0, CHUNK)],
                              sem).wait()

    def compute(p, g0):
        roff = p * CPW
        rows = [rv + roff for rv in rows0]
        coff = p * CHUNK

        @plsc.parallel_loop(0, CHUNK, step=LANES, unroll=4)
        def vec_body(col):
            sl = pl.ds(coff + col, LANES)
            li0 = y0buf[sl] - g0
            li1 = li0 + 1
            fv = fracbuf[sl]
            for ci in range(CPW):
                v0 = plsc.load_gather(xbuf, [rows[ci], li0])
                v1 = plsc.load_gather(xbuf, [rows[ci], li1])
                outbuf[roff + ci, pl.ds(col, LANES)] = v0 + fv * (v1 - v0)

    fire_in(0, 0, in_a)
    fire_in(1, 1, in_b)
    wait_in(0, in_a)
    compute(0, _g0_of(0))
    fire_in(2, 0, in_a)
    fire_out(0, 0, out_a)
    wait_in(1, in_b)
    compute(1, _g0_of(CHUNK))
    fire_in(3, 1, in_b)
    fire_out(1, 1, out_b)

    def chunk_body(k, _):
        p = k % 2
        in_sem = (in_a, in_b)
        def run(p_static):
            wait_out(p_static, out_sems[p_static])
            wait_in(p_static, in_sems[p_static])
            g0 = _g0_of(k * CHUNK)
            compute(p_static, g0)
            kn = jnp.minimum(k + 2, NFULL - 1)
            fire_in(kn, p_static, in_sems[p_static])
            fire_out(k, p_static, out_sems[p_static])

        lax.cond(p == 0, lambda: run(0), lambda: run(1))
        return 0

    lax.fori_loop(2, NFULL, chunk_body, 0)

    wait_out(1, out_b)
    wait_out(0, out_a)
    wait_in(1, in_b)
    wait_in(0, in_a)

    pltpu.sync_copy(y0_hbm.at[pl.ds(BULK, TAIL_OUT)], y0buf.at[pl.ds(0, TAIL_OUT)])
    pltpu.sync_copy(frac_hbm.at[pl.ds(BULK, TAIL_OUT)],
                    fracbuf.at[pl.ds(0, TAIL_OUT)])
    pltpu.sync_copy(x_hbm.at[pl.ds(cbase, CPW), pl.ds(TAIL_G0, TAIL_SPAN)],
                    xbuf.at[pl.ds(0, CPW), pl.ds(0, TAIL_SPAN)])
    li_max = IN_LEN - 1 - TAIL_G0

    @plsc.parallel_loop(0, TAIL_OUT, step=LANES, unroll=4)
    def tail_body(col):
        sl = pl.ds(col, LANES)
        li0 = y0buf[sl] - TAIL_G0
        li1 = jnp.minimum(li0 + 1, li_max)
        fv = fracbuf[sl]
        for ci in range(CPW):
            v0 = plsc.load_gather(xbuf, [rows0[ci], li0])
            v1 = plsc.load_gather(xbuf, [rows0[ci], li1])
            outbuf[ci, pl.ds(col, LANES)] = v0 + fv * (v1 - v0)

    pltpu.sync_copy(outbuf.at[pl.ds(0, CPW), pl.ds(0, TAIL_OUT)],
                    bulk_hbm.at[pl.ds(cbase, CPW), pl.ds(BULK, TAIL_OUT)])


def kernel(x, x_frac, y0_idx, y1_idx):
    pad = PAD_IDX - OUT_LEN
    y0p = jnp.pad(y0_idx, (0, pad), constant_values=IN_LEN - 2)
    fracp = jnp.pad(x_frac, (0, pad), constant_values=0.0)
    return _resample(x, y0p, fracp)[:, :OUT_LEN]


# speedup vs baseline: 1.8609x; 1.0387x over previous
import functools
import jax
import jax.numpy as jnp
from jax import lax
from jax.experimental import pallas as pl
from jax.experimental.pallas import tpu as pltpu
from jax.experimental.pallas import tpu_sc as plsc

IN_CH = 256
IN_LEN = 131072
OUT_LEN = 120423
CHUNK = 3072
NFULL = OUT_LEN // CHUNK
LANES = 16
NC, NS = 2, 16
NW = NC * NS
CPW = IN_CH // NW
SPAN = 3584


def _g0_of(j0):
    return (j0 + (j0 * 10649) // 120422) // 128 * 128


BULK = NFULL * CHUNK
TAIL_G0 = _g0_of(BULK)
TAIL_SPAN = IN_LEN - TAIL_G0
TAIL_OUT = 640
TAIL_NVEC = TAIL_OUT // LANES
PAD_IDX = BULK + TAIL_OUT


@functools.partial(
    pl.kernel,
    out_type=jax.ShapeDtypeStruct((IN_CH, PAD_IDX), jnp.float32),
    mesh=plsc.VectorSubcoreMesh(core_axis_name="c", subcore_axis_name="s"),
    scratch_types=[
        pltpu.VMEM((2 * CPW, SPAN), jnp.float32),
        pltpu.VMEM((2 * CHUNK,), jnp.int32),
        pltpu.VMEM((2 * CHUNK,), jnp.float32),
        pltpu.VMEM((2 * CPW, CHUNK), jnp.float32),
        pltpu.SemaphoreType.DMA,
        pltpu.SemaphoreType.DMA,
        pltpu.SemaphoreType.DMA,
        pltpu.SemaphoreType.DMA,
    ],
    compiler_params=pltpu.CompilerParams(use_tc_tiling_on_sc=True,
                                         needs_layout_passes=False),
)
def _resample(x_hbm, y0_hbm, frac_hbm, bulk_hbm, xbuf, y0buf,
              fracbuf, outbuf, in_a, in_b, out_a, out_b):
    wid = lax.axis_index("s") * NC + lax.axis_index("c")
    cbase = wid * CPW
    rows0 = [jnp.full((LANES,), ci, jnp.int32) for ci in range(CPW)]
    in_sems = (in_a, in_b)
    out_sems = (out_a, out_b)

    def fire_in(k, p, sem):
        j0 = k * CHUNK
        g0 = _g0_of(j0)
        pltpu.async_copy(y0_hbm.at[pl.ds(j0, CHUNK)],
                         y0buf.at[pl.ds(p * CHUNK, CHUNK)], sem)
        pltpu.async_copy(frac_hbm.at[pl.ds(j0, CHUNK)],
                         fracbuf.at[pl.ds(p * CHUNK, CHUNK)], sem)
        pltpu.async_copy(x_hbm.at[pl.ds(cbase, CPW), pl.ds(g0, SPAN)],
                         xbuf.at[pl.ds(p * CPW, CPW), :], sem)

    def wait_in(p, sem):
        pltpu.make_async_copy(y0_hbm.at[pl.ds(0, CHUNK)],
                              y0buf.at[pl.ds(p * CHUNK, CHUNK)], sem).wait()
        pltpu.make_async_copy(frac_hbm.at[pl.ds(0, CHUNK)],
                              fracbuf.at[pl.ds(p * CHUNK, CHUNK)], sem).wait()
        pltpu.make_async_copy(x_hbm.at[pl.ds(0, CPW), pl.ds(0, SPAN)],
                              xbuf.at[pl.ds(p * CPW, CPW), :], sem).wait()

    def fire_out(k, p, sem):
        pltpu.async_copy(outbuf.at[pl.ds(p * CPW, CPW), :],
                         bulk_hbm.at[pl.ds(cbase, CPW), pl.ds(k * CHUNK, CHUNK)],
                         sem)

    def wait_out(p, sem):
        pltpu.make_async_copy(outbuf.at[pl.ds(p * CPW, CPW), :],
                              bulk_hbm.at[pl.ds(cbase, CPW), pl.ds(0, CHUNK)],
                              sem).wait()

    def compute(p, g0):
        roff = p * CPW
        rows = [rv + roff for rv in rows0]
        coff = p * CHUNK

        @plsc.parallel_loop(0, CHUNK, step=LANES, unroll=2)
        def vec_body(col):
            sl = pl.ds(coff + col, LANES)
            li0 = y0buf[sl] - g0
            li1 = li0 + 1
            fv = fracbuf[sl]
            for ci in range(CPW):
                v0 = plsc.load_gather(xbuf, [rows[ci], li0])
                v1 = plsc.load_gather(xbuf, [rows[ci], li1])
                outbuf[roff + ci, pl.ds(col, LANES)] = v0 + fv * (v1 - v0)

    fire_in(0, 0, in_a)
    fire_in(1, 1, in_b)
    wait_in(0, in_a)
    compute(0, _g0_of(0))
    fire_in(2, 0, in_a)
    fire_out(0, 0, out_a)
    wait_in(1, in_b)
    compute(1, _g0_of(CHUNK))
    fire_in(3, 1, in_b)
    fire_out(1, 1, out_b)

    def chunk_body(k, _):
        p = k % 2
        in_sem = (in_a, in_b)
        def run(p_static):
            wait_out(p_static, out_sems[p_static])
            wait_in(p_static, in_sems[p_static])
            g0 = _g0_of(k * CHUNK)
            compute(p_static, g0)
            kn = jnp.minimum(k + 2, NFULL - 1)
            fire_in(kn, p_static, in_sems[p_static])
            fire_out(k, p_static, out_sems[p_static])

        lax.cond(p == 0, lambda: run(0), lambda: run(1))
        return 0

    lax.fori_loop(2, NFULL, chunk_body, 0)

    wait_out(1, out_b)
    wait_out(0, out_a)
    wait_in(1, in_b)
    wait_in(0, in_a)

    pltpu.sync_copy(y0_hbm.at[pl.ds(BULK, TAIL_OUT)], y0buf.at[pl.ds(0, TAIL_OUT)])
    pltpu.sync_copy(frac_hbm.at[pl.ds(BULK, TAIL_OUT)],
                    fracbuf.at[pl.ds(0, TAIL_OUT)])
    pltpu.sync_copy(x_hbm.at[pl.ds(cbase, CPW), pl.ds(TAIL_G0, TAIL_SPAN)],
                    xbuf.at[pl.ds(0, CPW), pl.ds(0, TAIL_SPAN)])
    li_max = IN_LEN - 1 - TAIL_G0

    @plsc.parallel_loop(0, TAIL_OUT, step=LANES, unroll=2)
    def tail_body(col):
        sl = pl.ds(col, LANES)
        li0 = y0buf[sl] - TAIL_G0
        li1 = jnp.minimum(li0 + 1, li_max)
        fv = fracbuf[sl]
        for ci in range(CPW):
            v0 = plsc.load_gather(xbuf, [rows0[ci], li0])
            v1 = plsc.load_gather(xbuf, [rows0[ci], li1])
            outbuf[ci, pl.ds(col, LANES)] = v0 + fv * (v1 - v0)

    pltpu.sync_copy(outbuf.at[pl.ds(0, CPW), pl.ds(0, TAIL_OUT)],
                    bulk_hbm.at[pl.ds(cbase, CPW), pl.ds(BULK, TAIL_OUT)])


def kernel(x, x_frac, y0_idx, y1_idx):
    pad = PAD_IDX - OUT_LEN
    y0p = jnp.pad(y0_idx, (0, pad), constant_values=IN_LEN - 2)
    fracp = jnp.pad(x_frac, (0, pad), constant_values=0.0)
    return _resample(x, y0p, fracp)[:, :OUT_LEN]


# speedup vs baseline: 1.8666x; 1.0030x over previous
import functools
import jax
import jax.numpy as jnp
from jax import lax
from jax.experimental import pallas as pl
from jax.experimental.pallas import tpu as pltpu
from jax.experimental.pallas import tpu_sc as plsc

IN_CH = 256
IN_LEN = 131072
OUT_LEN = 120423
CHUNK = 3072
NFULL = OUT_LEN // CHUNK
LANES = 16
NC, NS = 2, 16
NW = NC * NS
CPW = IN_CH // NW
SPAN = 3584


def _g0_of(j0):
    return (j0 + (j0 * 10649) // 120422) // 128 * 128


BULK = NFULL * CHUNK
TAIL_G0 = _g0_of(BULK)
TAIL_SPAN = IN_LEN - TAIL_G0
TAIL_OUT = 640
TAIL_NVEC = TAIL_OUT // LANES
PAD_IDX = BULK + TAIL_OUT


@functools.partial(
    pl.kernel,
    out_type=jax.ShapeDtypeStruct((IN_CH, PAD_IDX), jnp.float32),
    mesh=plsc.VectorSubcoreMesh(core_axis_name="c", subcore_axis_name="s"),
    scratch_types=[
        pltpu.VMEM((2 * CPW, SPAN), jnp.float32),
        pltpu.VMEM((2 * CHUNK,), jnp.int32),
        pltpu.VMEM((2 * CHUNK,), jnp.float32),
        pltpu.VMEM((2 * CPW, CHUNK), jnp.float32),
        pltpu.SemaphoreType.DMA,
        pltpu.SemaphoreType.DMA,
        pltpu.SemaphoreType.DMA,
        pltpu.SemaphoreType.DMA,
    ],
    compiler_params=pltpu.CompilerParams(use_tc_tiling_on_sc=True,
                                         needs_layout_passes=False),
)
def _resample(x_hbm, y0_hbm, frac_hbm, bulk_hbm, xbuf, y0buf,
              fracbuf, outbuf, in_a, in_b, out_a, out_b):
    wid = lax.axis_index("s") * NC + lax.axis_index("c")
    cbase = wid * CPW
    rows0 = [jnp.full((LANES,), ci, jnp.int32) for ci in range(CPW)]
    in_sems = (in_a, in_b)
    out_sems = (out_a, out_b)

    def fire_in(k, p, sem):
        j0 = k * CHUNK
        g0 = _g0_of(j0)
        pltpu.async_copy(y0_hbm.at[pl.ds(j0, CHUNK)],
                         y0buf.at[pl.ds(p * CHUNK, CHUNK)], sem)
        pltpu.async_copy(frac_hbm.at[pl.ds(j0, CHUNK)],
                         fracbuf.at[pl.ds(p * CHUNK, CHUNK)], sem)
        pltpu.async_copy(x_hbm.at[pl.ds(cbase, CPW), pl.ds(g0, SPAN)],
                         xbuf.at[pl.ds(p * CPW, CPW), :], sem)

    def wait_in(p, sem):
        pltpu.make_async_copy(y0_hbm.at[pl.ds(0, CHUNK)],
                              y0buf.at[pl.ds(p * CHUNK, CHUNK)], sem).wait()
        pltpu.make_async_copy(frac_hbm.at[pl.ds(0, CHUNK)],
                              fracbuf.at[pl.ds(p * CHUNK, CHUNK)], sem).wait()
        pltpu.make_async_copy(x_hbm.at[pl.ds(0, CPW), pl.ds(0, SPAN)],
                              xbuf.at[pl.ds(p * CPW, CPW), :], sem).wait()

    def fire_out(k, p, sem):
        pltpu.async_copy(outbuf.at[pl.ds(p * CPW, CPW), :],
                         bulk_hbm.at[pl.ds(cbase, CPW), pl.ds(k * CHUNK, CHUNK)],
                         sem)

    def wait_out(p, sem):
        pltpu.make_async_copy(outbuf.at[pl.ds(p * CPW, CPW), :],
                              bulk_hbm.at[pl.ds(cbase, CPW), pl.ds(0, CHUNK)],
                              sem).wait()

    def compute(p, g0):
        roff = p * CPW
        rows = [rv + roff for rv in rows0]
        coff = p * CHUNK

        @plsc.parallel_loop(0, CHUNK, step=LANES, unroll=1)
        def vec_body(col):
            sl = pl.ds(coff + col, LANES)
            li0 = y0buf[sl] - g0
            li1 = li0 + 1
            fv = fracbuf[sl]
            for ci in range(CPW):
                v0 = plsc.load_gather(xbuf, [rows[ci], li0])
                v1 = plsc.load_gather(xbuf, [rows[ci], li1])
                outbuf[roff + ci, pl.ds(col, LANES)] = v0 + fv * (v1 - v0)

    fire_in(0, 0, in_a)
    fire_in(1, 1, in_b)
    wait_in(0, in_a)
    compute(0, _g0_of(0))
    fire_in(2, 0, in_a)
    fire_out(0, 0, out_a)
    wait_in(1, in_b)
    compute(1, _g0_of(CHUNK))
    fire_in(3, 1, in_b)
    fire_out(1, 1, out_b)

    def chunk_body(k, _):
        p = k % 2
        in_sem = (in_a, in_b)
        def run(p_static):
            wait_out(p_static, out_sems[p_static])
            wait_in(p_static, in_sems[p_static])
            g0 = _g0_of(k * CHUNK)
            compute(p_static, g0)
            kn = jnp.minimum(k + 2, NFULL - 1)
            fire_in(kn, p_static, in_sems[p_static])
            fire_out(k, p_static, out_sems[p_static])

        lax.cond(p == 0, lambda: run(0), lambda: run(1))
        return 0

    lax.fori_loop(2, NFULL, chunk_body, 0)

    wait_out(1, out_b)
    wait_out(0, out_a)
    wait_in(1, in_b)
    wait_in(0, in_a)

    pltpu.sync_copy(y0_hbm.at[pl.ds(BULK, TAIL_OUT)], y0buf.at[pl.ds(0, TAIL_OUT)])
    pltpu.sync_copy(frac_hbm.at[pl.ds(BULK, TAIL_OUT)],
                    fracbuf.at[pl.ds(0, TAIL_OUT)])
    pltpu.sync_copy(x_hbm.at[pl.ds(cbase, CPW), pl.ds(TAIL_G0, TAIL_SPAN)],
                    xbuf.at[pl.ds(0, CPW), pl.ds(0, TAIL_SPAN)])
    li_max = IN_LEN - 1 - TAIL_G0

    @plsc.parallel_loop(0, TAIL_OUT, step=LANES, unroll=1)
    def tail_body(col):
        sl = pl.ds(col, LANES)
        li0 = y0buf[sl] - TAIL_G0
        li1 = jnp.minimum(li0 + 1, li_max)
        fv = fracbuf[sl]
        for ci in range(CPW):
            v0 = plsc.load_gather(xbuf, [rows0[ci], li0])
            v1 = plsc.load_gather(xbuf, [rows0[ci], li1])
            outbuf[ci, pl.ds(col, LANES)] = v0 + fv * (v1 - v0)

    pltpu.sync_copy(outbuf.at[pl.ds(0, CPW), pl.ds(0, TAIL_OUT)],
                    bulk_hbm.at[pl.ds(cbase, CPW), pl.ds(BULK, TAIL_OUT)])


def kernel(x, x_frac, y0_idx, y1_idx):
    pad = PAD_IDX - OUT_LEN
    y0p = jnp.pad(y0_idx, (0, pad), constant_values=IN_LEN - 2)
    fracp = jnp.pad(x_frac, (0, pad), constant_values=0.0)
    return _resample(x, y0p, fracp)[:, :OUT_LEN]
